# baseline (speedup 1.0000x reference)
import functools

import jax
import jax.numpy as jnp
from jax import lax
from jax.experimental import pallas as pl
from jax.experimental.pallas import tpu as pltpu
from jax.experimental.pallas import tpu_sc as plsc

H = 128
NC, NS, L = 2, 16, 16
NW = NC * NS
EDGE_BLK = 6400
GB = 64


def _dot16(a, w):
    return jnp.dot(a.astype(jnp.bfloat16), w.astype(jnp.bfloat16),
                   preferred_element_type=jnp.float32)


def _mm_relu_kernel(a_ref, w_ref, o_ref):
    o_ref[...] = jnp.maximum(_dot16(a_ref[...], w_ref[...]), 0.0)


def _mm_relu(a, w_t, blk):
    m, k = a.shape
    n = w_t.shape[1]
    return pl.pallas_call(
        _mm_relu_kernel,
        grid=(m // blk,),
        in_specs=[pl.BlockSpec((blk, k), lambda i: (i, 0)),
                  pl.BlockSpec((k, n), lambda i: (0, 0))],
        out_specs=pl.BlockSpec((blk, n), lambda i: (i, 0)),
        out_shape=jax.ShapeDtypeStruct((m, n), jnp.float32),
    )(a, w_t)


def _seg_body(npt, n_edges, x_hbm, edst_hbm, out_hbm,
              edst_v, pk_v, fids_v, rows_v, accs_v, accm_v, esem, gsem):
    wid = lax.axis_index("s") * NC + lax.axis_index("c")
    lo = wid * npt
    n_blk = n_edges // EDGE_BLK
    lanes = lax.iota(jnp.int32, L)

    zf = jnp.zeros((L,), jnp.float32)
    zi = jnp.zeros((L,), jnp.int32)

    def zero_acc(i, _):
        accs_v[pl.ds(i * L, L)] = zf
        accm_v[pl.ds(i * L, L)] = zf
        return 0
    lax.fori_loop(0, (npt + 1) * (H // L), zero_acc, 0)

    def zero_pk(i, _):
        pk_v[pl.ds(i * L, L)] = zi
        return 0
    lax.fori_loop(0, (EDGE_BLK + L) // L, zero_pk, 0)

    def edge_dma(b):
        par = b % 2
        return pltpu.make_async_copy(
            edst_hbm.at[pl.ds(b * EDGE_BLK, EDGE_BLK)], edst_v.at[par],
            esem.at[par])

    def fill_fids(i):
        par = (i % 2) * GB
        for g in range(GB // L):
            pk = pk_v[pl.ds(i * GB + g * L, L)]
            fids_v[pl.ds(par + g * L, L)] = pk & (2**19 - 1)

    def gather(i):
        par = (i % 2) * GB
        return pltpu.make_async_copy(
            x_hbm.at[fids_v.at[pl.ds(par, GB)]],
            rows_v.at[pl.ds(par, GB), :], gsem.at[i % 2])

    edge_dma(0).start()

    def blk_body(b, _):
        par_b = b % 2
        edge_dma(b).wait()

        @pl.when(b + 1 < n_blk)
        def _():
            edge_dma(b + 1).start()

        base = b * EDGE_BLK

        def scan_chunk(c, cnt):
            dst = edst_v[par_b, pl.ds(c * L, L)]
            msk = (dst >= lo) & (dst < lo + npt)
            pk = ((dst - lo) << 19) | (lanes + (base + c * L))
            plsc.store_compressed(pk_v.at[pl.ds(cnt, L)], pk, mask=msk)
            return cnt + plsc.all_reduce_population_count(msk)[0]
        cnt = lax.fori_loop(0, EDGE_BLK // L, scan_chunk, jnp.int32(0))

        nbat = (cnt + GB - 1) // GB

        @pl.when(nbat > 0)
        def _():
            fill_fids(0)
            gather(0).start()

        def bat_body(i, _):
            gather(i).wait()

            @pl.when(i + 1 < nbat)
            def _():
                fill_fids(i + 1)
                gather(i + 1).start()

            rbase = (i % 2) * GB
            nb = jnp.minimum(cnt - i * GB, GB)

            def acc_grp(jg, _):
                pk = pk_v[pl.ds(i * GB + jg * L, L)]
                dvec = pk >> 19
                idxv = lanes + jg * L
                dvec = jnp.where(idxv < nb, dvec, npt)
                for k16 in range(L):
                    off = dvec[k16] * H
                    j = rbase + jg * L + k16
                    for k in range(H // L):
                        r = rows_v[j, pl.ds(k * L, L)]
                        plsc.addupdate(accs_v.at[pl.ds(off + k * L, L)], r)
                        cur = accm_v[pl.ds(off + k * L, L)]
                        accm_v[pl.ds(off + k * L, L)] = jnp.maximum(cur, r)
                return 0
            lax.fori_loop(0, GB // L, acc_grp, 0)
            return 0
        lax.fori_loop(0, nbat, bat_body, 0)
        return 0

    lax.fori_loop(0, n_blk, blk_body, 0)

    def prod_row(i, _):
        accs_v[pl.ds(i * L, L)] = accs_v[pl.ds(i * L, L)] * accm_v[pl.ds(i * L, L)]
        return 0
    lax.fori_loop(0, npt * (H // L), prod_row, 0)
    pltpu.sync_copy(accs_v.at[pl.ds(0, npt * H)], out_hbm.at[pl.ds(lo * H, npt * H)])


def _segment_summax(x, edge_dst, n_pad):
    npt = n_pad // NW
    n_edges = x.shape[0]
    mesh = plsc.VectorSubcoreMesh(core_axis_name="c", subcore_axis_name="s",
                                  num_cores=NC, num_subcores=NS)
    body = functools.partial(_seg_body, npt, n_edges)
    out = pl.kernel(
        body,
        out_type=jax.ShapeDtypeStruct((n_pad * H,), jnp.float32),
        mesh=mesh,
        scratch_types=[
            pltpu.VMEM((2, EDGE_BLK), jnp.int32),
            pltpu.VMEM((EDGE_BLK + L,), jnp.int32),
            pltpu.VMEM((2 * GB,), jnp.int32),
            pltpu.VMEM((2 * GB, H), jnp.float32),
            pltpu.VMEM(((npt + 1) * H,), jnp.float32),
            pltpu.VMEM(((npt + 1) * H,), jnp.float32),
            pltpu.SemaphoreType.DMA((2,)),
            pltpu.SemaphoreType.DMA((2,)),
        ],
        compiler_params=pltpu.CompilerParams(needs_layout_passes=False),
    )(x, edge_dst)
    return out.reshape(n_pad, H)


def _gates_kernel(mg_ref, ia_ref, w1_ref, w2_ref, w3_ref, wif_ref, bif_ref,
                  wib_ref, bib_ref, h0_ref, gf_ref, gb_ref):
    mg = mg_ref[...]
    ia = ia_ref[...]
    f = ia + 2.0 * mg
    am = ((_dot16(mg, w1_ref[...]) + _dot16(f, w2_ref[...]))
          + _dot16(ia, w3_ref[...]))
    t = am.shape[0]
    h0_ref[...] = jnp.max(am, axis=0).reshape(1, 1, H)
    gf = _dot16(am, wif_ref[...]) + bif_ref[...]
    gb = _dot16(am, wib_ref[...]) + bib_ref[...]
    gf_ref[...] = gf.reshape(t, 1, 1, 3 * H)
    gb_ref[...] = gb.reshape(t, 1, 1, 3 * H)


def _gru_kernel(n_per, gf_ref, gb_ref, h0_ref, whf_ref, bhf_ref,
                whb_ref, bhb_ref, wof_ref, wob_ref, bo_ref, o_ref):
    b = h0_ref.shape[0]
    h0 = h0_ref[...]
    zero = jnp.zeros((b, H), jnp.float32)

    def gru_step(g, h, wh_ref, bh_ref):
        gh = _dot16(h, wh_ref[...]) + bh_ref[...]
        r = jax.nn.sigmoid(g[:, :H] + gh[:, :H])
        z = jax.nn.sigmoid(g[:, H:2 * H] + gh[:, H:2 * H])
        n = jnp.tanh(g[:, 2 * H:] + r * gh[:, 2 * H:])
        return (1.0 - z) * n + z * h

    def step(t, carry):
        hf, hb, sf, sb = carry
        hf = gru_step(gf_ref[t], hf, whf_ref, bhf_ref)
        hb = gru_step(gb_ref[n_per - 1 - t], hb, whb_ref, bhb_ref)
        return (hf, hb, sf + hf, sb + hb)

    _, _, sf, sb = lax.fori_loop(0, n_per, step, (h0, h0, zero, zero))
    inv = 1.0 / n_per
    emb = (_dot16(sf * inv, wof_ref[...]) + _dot16(sb * inv, wob_ref[...])
           + bo_ref[...])
    o_ref[...] = jnp.maximum(emb, 0.0)


def kernel(f_atoms, f_bonds, edge_src, edge_dst, W_i_atom, W_i_bond, W_h_0, W_h_1,
           W_lr, W_o, b_o, gru_Wih_f, gru_Whh_f, gru_bih_f, gru_bhh_f,
           gru_Wih_b, gru_Whh_b, gru_bih_b, gru_bhh_b):
    n_nodes = f_atoms.shape[0]
    n_graphs = 50
    n_per = n_nodes // n_graphs
    n_pad = ((n_nodes + NW - 1) // NW) * NW

    ia = _mm_relu(f_atoms, W_i_atom.T, 1000)
    x = _mm_relu(f_bonds, W_i_bond.T, 3200)

    messge = _segment_summax(x, edge_dst.astype(jnp.int32), n_pad)

    h0, gf, gb = pl.pallas_call(
        _gates_kernel,
        grid=(n_graphs,),
        in_specs=[
            pl.BlockSpec((n_per, H), lambda i: (i, 0)),
            pl.BlockSpec((n_per, H), lambda i: (i, 0)),
            pl.BlockSpec((H, H), lambda i: (0, 0)),
            pl.BlockSpec((H, H), lambda i: (0, 0)),
            pl.BlockSpec((H, H), lambda i: (0, 0)),
            pl.BlockSpec((H, 3 * H), lambda i: (0, 0)),
            pl.BlockSpec((1, 3 * H), lambda i: (0, 0)),
            pl.BlockSpec((H, 3 * H), lambda i: (0, 0)),
            pl.BlockSpec((1, 3 * H), lambda i: (0, 0)),
        ],
        out_specs=[
            pl.BlockSpec((1, 1, H), lambda i: (i, 0, 0)),
            pl.BlockSpec((n_per, 1, 1, 3 * H), lambda i: (0, i, 0, 0)),
            pl.BlockSpec((n_per, 1, 1, 3 * H), lambda i: (0, i, 0, 0)),
        ],
        out_shape=[
            jax.ShapeDtypeStruct((n_graphs, 1, H), jnp.float32),
            jax.ShapeDtypeStruct((n_per, n_graphs, 1, 3 * H), jnp.float32),
            jax.ShapeDtypeStruct((n_per, n_graphs, 1, 3 * H), jnp.float32),
        ],
    )(messge, ia, W_lr[:, :H].T, W_lr[:, H:2 * H].T, W_lr[:, 2 * H:].T,
      gru_Wih_f.T, gru_bih_f.reshape(1, 3 * H),
      gru_Wih_b.T, gru_bih_b.reshape(1, 3 * H))

    out = pl.pallas_call(
        functools.partial(_gru_kernel, n_per),
        out_shape=jax.ShapeDtypeStruct((n_graphs, H), jnp.float32),
    )(gf.reshape(n_per, n_graphs, 3 * H), gb.reshape(n_per, n_graphs, 3 * H),
      h0.reshape(n_graphs, H), gru_Whh_f.T, gru_bhh_f.reshape(1, 3 * H),
      gru_Whh_b.T, gru_bhh_b.reshape(1, 3 * H),
      W_o[:, :H].T, W_o[:, H:].T, b_o.reshape(1, H))
    return out


# speedup vs baseline: 1.3451x; 1.3451x over previous
import functools

import jax
import jax.numpy as jnp
from jax import lax
from jax.experimental import pallas as pl
from jax.experimental.pallas import tpu as pltpu
from jax.experimental.pallas import tpu_sc as plsc

H = 128
NC, NS, L = 2, 16, 16
NW = NC * NS
EDGE_BLK = 3200
GB = 96
NSLOT = 3
PK_CAP = 128


def _dot16(a, w):
    return jnp.dot(a.astype(jnp.bfloat16), w.astype(jnp.bfloat16),
                   preferred_element_type=jnp.float32)


def _mm_relu_kernel(a_ref, w_ref, o_ref):
    o_ref[...] = jnp.maximum(_dot16(a_ref[...], w_ref[...]), 0.0)


def _mm_relu(a, w_t, blk):
    m, k = a.shape
    n = w_t.shape[1]
    return pl.pallas_call(
        _mm_relu_kernel,
        grid=(m // blk,),
        in_specs=[pl.BlockSpec((blk, k), lambda i: (i, 0)),
                  pl.BlockSpec((k, n), lambda i: (0, 0))],
        out_specs=pl.BlockSpec((blk, n), lambda i: (i, 0)),
        out_shape=jax.ShapeDtypeStruct((m, n), jnp.float32),
    )(a, w_t)


def _seg_body(npt, n_pad, n_edges, x_hbm, edst_hbm, m_hbm, s_hbm,
              edst_v, pk_v, fids_v, fdst_v, rows_v, accm_v, s_sh, esem, gsem):
    cid = lax.axis_index("c")
    sid = lax.axis_index("s")
    wid = cid * NS + sid
    lo = wid * npt
    sloc = sid * npt
    cbase = cid * NS * npt
    n_blk = n_edges // EDGE_BLK
    n_ch = (NS * npt) // L
    lanes = lax.iota(jnp.int32, L)

    zf = jnp.zeros((L,), jnp.float32)
    zi = jnp.zeros((L,), jnp.int32)

    def zero_acc(i, _):
        accm_v[pl.ds(i * L, L)] = zf
        return 0
    lax.fori_loop(0, (npt + 1) * (H // L), zero_acc, 0)

    def zero_pk(i, _):
        pk_v[pl.ds(i * L, L)] = zi
        return 0
    lax.fori_loop(0, PK_CAP // L, zero_pk, 0)
    for s_ in range(NSLOT):
        for g in range(GB // L):
            fids_v[s_, pl.ds(g * L, L)] = zi
            fdst_v[s_, pl.ds(g * L, L)] = zi

    def zero_rows(i, _):
        for k in range(H // L):
            rows_v[i, pl.ds(k * L, L)] = zf
        return 0
    lax.fori_loop(0, L, zero_rows, 0)

    def zero_chunk(i, _):
        c = sid + NS * i

        @pl.when(c < n_ch)
        def _():
            pltpu.sync_copy(rows_v.at[pl.ds(0, L), :],
                            s_sh.at[pl.ds(c * L, L), :])
        return 0
    lax.fori_loop(0, (n_ch + NS - 1) // NS, zero_chunk, 0)
    plsc.subcore_barrier()

    def edge_dma(b):
        par = b % 2
        return pltpu.make_async_copy(
            edst_hbm.at[pl.ds(b * EDGE_BLK, EDGE_BLK)], edst_v.at[par],
            esem.at[par])

    def fill_slot(nf):
        s_ = nf % NSLOT
        for g in range(GB // L):
            pk = pk_v[pl.ds(g * L, L)]
            fids_v[s_, pl.ds(g * L, L)] = pk & (2**19 - 1)
            fdst_v[s_, pl.ds(g * L, L)] = (pk >> 19) + sloc
        for g in range((PK_CAP - GB) // L):
            pk_v[pl.ds(g * L, L)] = pk_v[pl.ds(GB + g * L, L)]

    def gather_start(nf):
        s_ = nf % NSLOT
        pltpu.async_copy(x_hbm.at[fids_v.at[s_]],
                         rows_v.at[pl.ds(s_ * GB, GB), :], gsem.at[s_])

    def finish(nf, nb):
        s_ = nf % NSLOT
        pltpu.make_async_copy(x_hbm.at[fids_v.at[s_]],
                              rows_v.at[pl.ds(s_ * GB, GB), :],
                              gsem.at[s_]).wait()

        def zrow(j, _):
            for k in range(H // L):
                rows_v[s_ * GB + j, pl.ds(k * L, L)] = zf
            return 0
        lax.fori_loop(nb, GB, zrow, 0)

        def acc_grp(jg, _):
            dg = fdst_v[s_, pl.ds(jg * L, L)]
            dvec = jnp.where(lanes + jg * L < nb, dg - sloc, npt)
            for k16 in range(L):
                off = dvec[k16] * H
                j = s_ * GB + jg * L + k16
                for k in range(H // L):
                    r = rows_v[j, pl.ds(k * L, L)]
                    cur = accm_v[pl.ds(off + k * L, L)]
                    accm_v[pl.ds(off + k * L, L)] = jnp.maximum(cur, r)
            return 0
        lax.fori_loop(0, GB // L, acc_grp, 0)
        pltpu.sync_copy(rows_v.at[pl.ds(s_ * GB, GB), :],
                        s_sh.at[fdst_v.at[s_]], add=True)

    edge_dma(0).start()

    def blk_body(b, carry):
        par_b = b % 2
        edge_dma(b).wait()

        @pl.when(b + 1 < n_blk)
        def _():
            edge_dma(b + 1).start()

        base = b * EDGE_BLK

        def chunk(c, carry):
            cnt, nf = carry
            dst = edst_v[par_b, pl.ds(c * L, L)]
            msk = (dst >= lo) & (dst < lo + npt)
            pk = ((dst - lo) << 19) | (lanes + (base + c * L))
            plsc.store_compressed(pk_v.at[pl.ds(cnt, L)], pk, mask=msk)
            cnt = cnt + plsc.all_reduce_population_count(msk)[0]

            def flush(args):
                cnt, nf = args
                fill_slot(nf)
                gather_start(nf)

                @pl.when(nf >= 2)
                def _():
                    finish(nf - 2, GB)
                return (cnt - GB, nf + 1)
            return lax.cond(cnt >= GB, flush, lambda a: a, (cnt, nf))
        return lax.fori_loop(0, EDGE_BLK // L, chunk, carry)

    cnt, nf = lax.fori_loop(0, n_blk, blk_body,
                            (jnp.int32(0), jnp.int32(0)))

    @pl.when(nf >= 2)
    def _():
        finish(nf - 2, GB)

    @pl.when(nf >= 1)
    def _():
        finish(nf - 1, GB)

    @pl.when(cnt > 0)
    def _():
        fill_slot(nf)
        gather_start(nf)
        finish(nf, cnt)

    plsc.subcore_barrier()
    pltpu.sync_copy(accm_v.at[pl.ds(0, npt * H)],
                    m_hbm.at[pl.ds(lo * H, npt * H)])

    def out_chunk(i, _):
        c = sid + NS * i

        @pl.when(c < n_ch)
        def _():
            pltpu.sync_copy(s_sh.at[pl.ds(c * L, L), :],
                            s_hbm.at[pl.ds(cbase + c * L, L), :])
        return 0
    lax.fori_loop(0, (n_ch + NS - 1) // NS, out_chunk, 0)


def _segment_summax(x, edge_dst, n_pad):
    npt = n_pad // NW
    n_edges = x.shape[0]
    mesh = plsc.VectorSubcoreMesh(core_axis_name="c", subcore_axis_name="s",
                                  num_cores=NC, num_subcores=NS)
    body = functools.partial(_seg_body, npt, n_pad, n_edges)
    m_flat, s = pl.kernel(
        body,
        out_type=[jax.ShapeDtypeStruct((n_pad * H,), jnp.float32),
                  jax.ShapeDtypeStruct((n_pad, H), jnp.float32)],
        mesh=mesh,
        scratch_types=[
            pltpu.VMEM((2, EDGE_BLK), jnp.int32),
            pltpu.VMEM((PK_CAP,), jnp.int32),
            pltpu.VMEM((NSLOT, GB), jnp.int32),
            pltpu.VMEM((NSLOT, GB), jnp.int32),
            pltpu.VMEM((NSLOT * GB, H), jnp.float32),
            pltpu.VMEM(((npt + 1) * H,), jnp.float32),
            pltpu.VMEM_SHARED((NS * npt, H), jnp.float32),
            pltpu.SemaphoreType.DMA((2,)),
            pltpu.SemaphoreType.DMA((NSLOT,)),
        ],
        compiler_params=pltpu.CompilerParams(needs_layout_passes=False),
    )(x, edge_dst)
    return m_flat.reshape(n_pad, H), s


def _gates_kernel(m_ref, s_ref, ia_ref, w1_ref, w2_ref, w3_ref, wif_ref, bif_ref,
                  wib_ref, bib_ref, h0_ref, gf_ref, gb_ref):
    mg = s_ref[...] * m_ref[...]
    ia = ia_ref[...]
    f = ia + 2.0 * mg
    am = ((_dot16(mg, w1_ref[...]) + _dot16(f, w2_ref[...]))
          + _dot16(ia, w3_ref[...]))
    t = am.shape[0]
    h0_ref[...] = jnp.max(am, axis=0).reshape(1, 1, H)
    gf = _dot16(am, wif_ref[...]) + bif_ref[...]
    gb = _dot16(am, wib_ref[...]) + bib_ref[...]
    gf_ref[...] = gf.reshape(t, 1, 1, 3 * H)
    gb_ref[...] = gb.reshape(t, 1, 1, 3 * H)


def _gru_kernel(n_per, gf_ref, gb_ref, h0_ref, whf_ref, bhf_ref,
                whb_ref, bhb_ref, wof_ref, wob_ref, bo_ref, o_ref):
    b = h0_ref.shape[0]
    h0 = h0_ref[...]
    zero = jnp.zeros((b, H), jnp.float32)

    def gru_step(g, h, wh_ref, bh_ref):
        gh = _dot16(h, wh_ref[...]) + bh_ref[...]
        r = jax.nn.sigmoid(g[:, :H] + gh[:, :H])
        z = jax.nn.sigmoid(g[:, H:2 * H] + gh[:, H:2 * H])
        n = jnp.tanh(g[:, 2 * H:] + r * gh[:, 2 * H:])
        return (1.0 - z) * n + z * h

    def step(t, carry):
        hf, hb, sf, sb = carry
        hf = gru_step(gf_ref[t], hf, whf_ref, bhf_ref)
        hb = gru_step(gb_ref[n_per - 1 - t], hb, whb_ref, bhb_ref)
        return (hf, hb, sf + hf, sb + hb)

    _, _, sf, sb = lax.fori_loop(0, n_per, step, (h0, h0, zero, zero))
    inv = 1.0 / n_per
    emb = (_dot16(sf * inv, wof_ref[...]) + _dot16(sb * inv, wob_ref[...])
           + bo_ref[...])
    o_ref[...] = jnp.maximum(emb, 0.0)


def kernel(f_atoms, f_bonds, edge_src, edge_dst, W_i_atom, W_i_bond, W_h_0, W_h_1,
           W_lr, W_o, b_o, gru_Wih_f, gru_Whh_f, gru_bih_f, gru_bhh_f,
           gru_Wih_b, gru_Whh_b, gru_bih_b, gru_bhh_b):
    n_nodes = f_atoms.shape[0]
    n_graphs = 50
    n_per = n_nodes // n_graphs
    n_pad = ((n_nodes + NW - 1) // NW) * NW

    ia = _mm_relu(f_atoms, W_i_atom.T, 1000)
    x = _mm_relu(f_bonds, W_i_bond.T, 3200)

    m, s = _segment_summax(x, edge_dst.astype(jnp.int32), n_pad)

    h0, gf, gb = pl.pallas_call(
        _gates_kernel,
        grid=(n_graphs,),
        in_specs=[
            pl.BlockSpec((n_per, H), lambda i: (i, 0)),
            pl.BlockSpec((n_per, H), lambda i: (i, 0)),
            pl.BlockSpec((n_per, H), lambda i: (i, 0)),
            pl.BlockSpec((H, H), lambda i: (0, 0)),
            pl.BlockSpec((H, H), lambda i: (0, 0)),
            pl.BlockSpec((H, H), lambda i: (0, 0)),
            pl.BlockSpec((H, 3 * H), lambda i: (0, 0)),
            pl.BlockSpec((1, 3 * H), lambda i: (0, 0)),
            pl.BlockSpec((H, 3 * H), lambda i: (0, 0)),
            pl.BlockSpec((1, 3 * H), lambda i: (0, 0)),
        ],
        out_specs=[
            pl.BlockSpec((1, 1, H), lambda i: (i, 0, 0)),
            pl.BlockSpec((n_per, 1, 1, 3 * H), lambda i: (0, i, 0, 0)),
            pl.BlockSpec((n_per, 1, 1, 3 * H), lambda i: (0, i, 0, 0)),
        ],
        out_shape=[
            jax.ShapeDtypeStruct((n_graphs, 1, H), jnp.float32),
            jax.ShapeDtypeStruct((n_per, n_graphs, 1, 3 * H), jnp.float32),
            jax.ShapeDtypeStruct((n_per, n_graphs, 1, 3 * H), jnp.float32),
        ],
    )(m, s, ia, W_lr[:, :H].T, W_lr[:, H:2 * H].T, W_lr[:, 2 * H:].T,
      gru_Wih_f.T, gru_bih_f.reshape(1, 3 * H),
      gru_Wih_b.T, gru_bih_b.reshape(1, 3 * H))

    out = pl.pallas_call(
        functools.partial(_gru_kernel, n_per),
        out_shape=jax.ShapeDtypeStruct((n_graphs, H), jnp.float32),
    )(gf.reshape(n_per, n_graphs, 3 * H), gb.reshape(n_per, n_graphs, 3 * H),
      h0.reshape(n_graphs, H), gru_Whh_f.T, gru_bhh_f.reshape(1, 3 * H),
      gru_Whh_b.T, gru_bhh_b.reshape(1, 3 * H),
      W_o[:, :H].T, W_o[:, H:].T, b_o.reshape(1, H))
    return out


# speedup vs baseline: 1.3962x; 1.0380x over previous
import functools

import jax
import jax.numpy as jnp
from jax import lax
from jax.experimental import pallas as pl
from jax.experimental.pallas import tpu as pltpu
from jax.experimental.pallas import tpu_sc as plsc

H = 128
NC, NS, L = 2, 16, 16
NW = NC * NS
EDGE_BLK = 3200
GB = 96
NSLOT = 3
PK_CAP = 128


def _dot16(a, w):
    return jnp.dot(a.astype(jnp.bfloat16), w.astype(jnp.bfloat16),
                   preferred_element_type=jnp.float32)


def _mm_relu_kernel(a_ref, w_ref, o_ref):
    o_ref[...] = jnp.maximum(_dot16(a_ref[...], w_ref[...]), 0.0)


def _mm_relu(a, w_t, blk):
    m, k = a.shape
    n = w_t.shape[1]
    return pl.pallas_call(
        _mm_relu_kernel,
        grid=(m // blk,),
        in_specs=[pl.BlockSpec((blk, k), lambda i: (i, 0)),
                  pl.BlockSpec((k, n), lambda i: (0, 0))],
        out_specs=pl.BlockSpec((blk, n), lambda i: (i, 0)),
        out_shape=jax.ShapeDtypeStruct((m, n), jnp.float32),
    )(a, w_t)


def _seg_body(npt, n_pad, n_edges, x_hbm, edst_hbm, m_hbm, s_hbm,
              edst_v, pk_v, fids_v, fdst_v, rows_v, accm_v, s_sh, esem, gsem,
              ssem):
    cid = lax.axis_index("c")
    sid = lax.axis_index("s")
    wid = cid * NS + sid
    lo = wid * npt
    sloc = sid * npt
    cbase = cid * NS * npt
    n_blk = n_edges // EDGE_BLK
    n_ch = (NS * npt) // L
    lanes = lax.iota(jnp.int32, L)

    zf = jnp.zeros((L,), jnp.float32)
    zi = jnp.zeros((L,), jnp.int32)

    def zero_acc(i, _):
        accm_v[pl.ds(i * L, L)] = zf
        return 0
    lax.fori_loop(0, (npt + 1) * (H // L), zero_acc, 0)

    def zero_pk(i, _):
        pk_v[pl.ds(i * L, L)] = zi
        return 0
    lax.fori_loop(0, PK_CAP // L, zero_pk, 0)
    for s_ in range(NSLOT):
        for g in range(GB // L):
            fids_v[s_, pl.ds(g * L, L)] = zi
            fdst_v[s_, pl.ds(g * L, L)] = zi

    def zero_rows(i, _):
        for k in range(H // L):
            rows_v[i, pl.ds(k * L, L)] = zf
        return 0
    lax.fori_loop(0, L, zero_rows, 0)

    def zero_chunk(i, _):
        c = sid + NS * i

        @pl.when(c < n_ch)
        def _():
            pltpu.sync_copy(rows_v.at[pl.ds(0, L), :],
                            s_sh.at[pl.ds(c * L, L), :])
        return 0
    lax.fori_loop(0, (n_ch + NS - 1) // NS, zero_chunk, 0)
    plsc.subcore_barrier()

    def edge_dma(b):
        par = b % 2
        return pltpu.make_async_copy(
            edst_hbm.at[pl.ds(b * EDGE_BLK, EDGE_BLK)], edst_v.at[par],
            esem.at[par])

    def fill_slot(nf):
        s_ = nf % NSLOT
        for g in range(GB // L):
            pk = pk_v[pl.ds(g * L, L)]
            fids_v[s_, pl.ds(g * L, L)] = pk & (2**19 - 1)
            fdst_v[s_, pl.ds(g * L, L)] = (pk >> 19) + sloc
        for g in range((PK_CAP - GB) // L):
            pk_v[pl.ds(g * L, L)] = pk_v[pl.ds(GB + g * L, L)]

    def gather_start(nf):
        s_ = nf % NSLOT
        pltpu.async_copy(x_hbm.at[fids_v.at[s_]],
                         rows_v.at[pl.ds(s_ * GB, GB), :], gsem.at[s_])

    def finish(nf, nb):
        s_ = nf % NSLOT
        pltpu.make_async_copy(x_hbm.at[fids_v.at[s_]],
                              rows_v.at[pl.ds(s_ * GB, GB), :],
                              gsem.at[s_]).wait()

        def zrow(j, _):
            for k in range(H // L):
                rows_v[s_ * GB + j, pl.ds(k * L, L)] = zf
            return 0
        lax.fori_loop(nb, GB, zrow, 0)

        def acc_grp(jg, _):
            dg = fdst_v[s_, pl.ds(jg * L, L)]
            dvec = jnp.where(lanes + jg * L < nb, dg - sloc, npt)
            for k16 in range(L):
                off = dvec[k16] * H
                j = s_ * GB + jg * L + k16
                for k in range(H // L):
                    r = rows_v[j, pl.ds(k * L, L)]
                    cur = accm_v[pl.ds(off + k * L, L)]
                    accm_v[pl.ds(off + k * L, L)] = jnp.maximum(cur, r)
            return 0
        lax.fori_loop(0, GB // L, acc_grp, 0)
        pltpu.async_copy(rows_v.at[pl.ds(s_ * GB, GB), :],
                         s_sh.at[fdst_v.at[s_]], ssem.at[s_], add=True)

    def scat_wait(s_):
        pltpu.make_async_copy(rows_v.at[pl.ds(s_ * GB, GB), :],
                              s_sh.at[fdst_v.at[s_]], ssem.at[s_]).wait()

    edge_dma(0).start()

    def blk_body(b, carry):
        par_b = b % 2
        edge_dma(b).wait()

        @pl.when(b + 1 < n_blk)
        def _():
            edge_dma(b + 1).start()

        base = b * EDGE_BLK

        def chunk(c, carry):
            cnt, nf = carry
            dst = edst_v[par_b, pl.ds(c * L, L)]
            msk = (dst >= lo) & (dst < lo + npt)
            pk = ((dst - lo) << 19) | (lanes + (base + c * L))
            plsc.store_compressed(pk_v.at[pl.ds(cnt, L)], pk, mask=msk)
            cnt = cnt + plsc.all_reduce_population_count(msk)[0]

            def flush(args):
                cnt, nf = args

                @pl.when(nf >= NSLOT)
                def _():
                    scat_wait(nf % NSLOT)
                fill_slot(nf)
                gather_start(nf)

                @pl.when(nf >= 2)
                def _():
                    finish(nf - 2, GB)
                return (cnt - GB, nf + 1)
            return lax.cond(cnt >= GB, flush, lambda a: a, (cnt, nf))
        return lax.fori_loop(0, EDGE_BLK // L, chunk, carry)

    cnt, nf = lax.fori_loop(0, n_blk, blk_body,
                            (jnp.int32(0), jnp.int32(0)))

    @pl.when(nf >= 2)
    def _():
        finish(nf - 2, GB)

    @pl.when(nf >= 1)
    def _():
        finish(nf - 1, GB)

    @pl.when(cnt > 0)
    def _():
        @pl.when(nf >= NSLOT)
        def _():
            scat_wait(nf % NSLOT)
        fill_slot(nf)
        gather_start(nf)
        finish(nf, cnt)

    tf = jnp.where(cnt > 0, nf + 1, nf)
    for k in range(1, NSLOT + 1):
        @pl.when(tf >= k)
        def _(k=k):
            scat_wait((tf - k) % NSLOT)

    plsc.subcore_barrier()
    pltpu.sync_copy(accm_v.at[pl.ds(0, npt * H)],
                    m_hbm.at[pl.ds(lo * H, npt * H)])

    def out_chunk(i, _):
        c = sid + NS * i

        @pl.when(c < n_ch)
        def _():
            pltpu.sync_copy(s_sh.at[pl.ds(c * L, L), :],
                            s_hbm.at[pl.ds(cbase + c * L, L), :])
        return 0
    lax.fori_loop(0, (n_ch + NS - 1) // NS, out_chunk, 0)


def _segment_summax(x, edge_dst, n_pad):
    npt = n_pad // NW
    n_edges = x.shape[0]
    mesh = plsc.VectorSubcoreMesh(core_axis_name="c", subcore_axis_name="s",
                                  num_cores=NC, num_subcores=NS)
    body = functools.partial(_seg_body, npt, n_pad, n_edges)
    m_flat, s = pl.kernel(
        body,
        out_type=[jax.ShapeDtypeStruct((n_pad * H,), jnp.float32),
                  jax.ShapeDtypeStruct((n_pad, H), jnp.float32)],
        mesh=mesh,
        scratch_types=[
            pltpu.VMEM((2, EDGE_BLK), jnp.int32),
            pltpu.VMEM((PK_CAP,), jnp.int32),
            pltpu.VMEM((NSLOT, GB), jnp.int32),
            pltpu.VMEM((NSLOT, GB), jnp.int32),
            pltpu.VMEM((NSLOT * GB, H), jnp.float32),
            pltpu.VMEM(((npt + 1) * H,), jnp.float32),
            pltpu.VMEM_SHARED((NS * npt, H), jnp.float32),
            pltpu.SemaphoreType.DMA((2,)),
            pltpu.SemaphoreType.DMA((NSLOT,)),
            pltpu.SemaphoreType.DMA((NSLOT,)),
        ],
        compiler_params=pltpu.CompilerParams(needs_layout_passes=False),
    )(x, edge_dst)
    return m_flat.reshape(n_pad, H), s


def _gates_kernel(m_ref, s_ref, ia_ref, w1_ref, w2_ref, w3_ref, wif_ref, bif_ref,
                  wib_ref, bib_ref, h0_ref, gf_ref, gb_ref):
    mg = s_ref[...] * m_ref[...]
    ia = ia_ref[...]
    f = ia + 2.0 * mg
    am = ((_dot16(mg, w1_ref[...]) + _dot16(f, w2_ref[...]))
          + _dot16(ia, w3_ref[...]))
    t = am.shape[0]
    h0_ref[...] = jnp.max(am, axis=0).reshape(1, 1, H)
    gf = _dot16(am, wif_ref[...]) + bif_ref[...]
    gb = _dot16(am, wib_ref[...]) + bib_ref[...]
    gf_ref[...] = gf.reshape(t, 1, 1, 3 * H)
    gb_ref[...] = gb.reshape(t, 1, 1, 3 * H)


def _gru_kernel(n_per, gf_ref, gb_ref, h0_ref, whf_ref, bhf_ref,
                whb_ref, bhb_ref, wof_ref, wob_ref, bo_ref, o_ref):
    b = h0_ref.shape[0]
    h0 = h0_ref[...]
    zero = jnp.zeros((b, H), jnp.float32)

    def gru_step(g, h, wh_ref, bh_ref):
        gh = _dot16(h, wh_ref[...]) + bh_ref[...]
        r = jax.nn.sigmoid(g[:, :H] + gh[:, :H])
        z = jax.nn.sigmoid(g[:, H:2 * H] + gh[:, H:2 * H])
        n = jnp.tanh(g[:, 2 * H:] + r * gh[:, 2 * H:])
        return (1.0 - z) * n + z * h

    def step(t, carry):
        hf, hb, sf, sb = carry
        hf = gru_step(gf_ref[t], hf, whf_ref, bhf_ref)
        hb = gru_step(gb_ref[n_per - 1 - t], hb, whb_ref, bhb_ref)
        return (hf, hb, sf + hf, sb + hb)

    _, _, sf, sb = lax.fori_loop(0, n_per, step, (h0, h0, zero, zero))
    inv = 1.0 / n_per
    emb = (_dot16(sf * inv, wof_ref[...]) + _dot16(sb * inv, wob_ref[...])
           + bo_ref[...])
    o_ref[...] = jnp.maximum(emb, 0.0)


def kernel(f_atoms, f_bonds, edge_src, edge_dst, W_i_atom, W_i_bond, W_h_0, W_h_1,
           W_lr, W_o, b_o, gru_Wih_f, gru_Whh_f, gru_bih_f, gru_bhh_f,
           gru_Wih_b, gru_Whh_b, gru_bih_b, gru_bhh_b):
    n_nodes = f_atoms.shape[0]
    n_graphs = 50
    n_per = n_nodes // n_graphs
    n_pad = ((n_nodes + NW - 1) // NW) * NW

    ia = _mm_relu(f_atoms, W_i_atom.T, 1000)
    x = _mm_relu(f_bonds, W_i_bond.T, 3200)

    m, s = _segment_summax(x, edge_dst.astype(jnp.int32), n_pad)

    h0, gf, gb = pl.pallas_call(
        _gates_kernel,
        grid=(n_graphs,),
        in_specs=[
            pl.BlockSpec((n_per, H), lambda i: (i, 0)),
            pl.BlockSpec((n_per, H), lambda i: (i, 0)),
            pl.BlockSpec((n_per, H), lambda i: (i, 0)),
            pl.BlockSpec((H, H), lambda i: (0, 0)),
            pl.BlockSpec((H, H), lambda i: (0, 0)),
            pl.BlockSpec((H, H), lambda i: (0, 0)),
            pl.BlockSpec((H, 3 * H), lambda i: (0, 0)),
            pl.BlockSpec((1, 3 * H), lambda i: (0, 0)),
            pl.BlockSpec((H, 3 * H), lambda i: (0, 0)),
            pl.BlockSpec((1, 3 * H), lambda i: (0, 0)),
        ],
        out_specs=[
            pl.BlockSpec((1, 1, H), lambda i: (i, 0, 0)),
            pl.BlockSpec((n_per, 1, 1, 3 * H), lambda i: (0, i, 0, 0)),
            pl.BlockSpec((n_per, 1, 1, 3 * H), lambda i: (0, i, 0, 0)),
        ],
        out_shape=[
            jax.ShapeDtypeStruct((n_graphs, 1, H), jnp.float32),
            jax.ShapeDtypeStruct((n_per, n_graphs, 1, 3 * H), jnp.float32),
            jax.ShapeDtypeStruct((n_per, n_graphs, 1, 3 * H), jnp.float32),
        ],
    )(m, s, ia, W_lr[:, :H].T, W_lr[:, H:2 * H].T, W_lr[:, 2 * H:].T,
      gru_Wih_f.T, gru_bih_f.reshape(1, 3 * H),
      gru_Wih_b.T, gru_bih_b.reshape(1, 3 * H))

    out = pl.pallas_call(
        functools.partial(_gru_kernel, n_per),
        out_shape=jax.ShapeDtypeStruct((n_graphs, H), jnp.float32),
    )(gf.reshape(n_per, n_graphs, 3 * H), gb.reshape(n_per, n_graphs, 3 * H),
      h0.reshape(n_graphs, H), gru_Whh_f.T, gru_bhh_f.reshape(1, 3 * H),
      gru_Whh_b.T, gru_bhh_b.reshape(1, 3 * H),
      W_o[:, :H].T, W_o[:, H:].T, b_o.reshape(1, H))
    return out


# speedup vs baseline: 1.4226x; 1.0189x over previous
import functools

import jax
import jax.numpy as jnp
from jax import lax
from jax.experimental import pallas as pl
from jax.experimental.pallas import tpu as pltpu
from jax.experimental.pallas import tpu_sc as plsc

H = 128
NC, NS, L = 2, 16, 16
NW = NC * NS
EDGE_BLK = 3200
GB = 96
NSLOT = 3
PK_CAP = 128


def _dot16(a, w):
    return jnp.dot(a.astype(jnp.bfloat16), w.astype(jnp.bfloat16),
                   preferred_element_type=jnp.float32)


def _mm_relu_kernel(a_ref, w_ref, o_ref):
    o_ref[...] = jnp.maximum(_dot16(a_ref[...], w_ref[...]), 0.0)


def _mm_relu(a, w_t, blk):
    m, k = a.shape
    n = w_t.shape[1]
    return pl.pallas_call(
        _mm_relu_kernel,
        grid=(m // blk,),
        in_specs=[pl.BlockSpec((blk, k), lambda i: (i, 0)),
                  pl.BlockSpec((k, n), lambda i: (0, 0))],
        out_specs=pl.BlockSpec((blk, n), lambda i: (i, 0)),
        out_shape=jax.ShapeDtypeStruct((m, n), jnp.float32),
    )(a, w_t)


def _seg_body(npt, n_pad, n_edges, x_hbm, edst_hbm, m_hbm, s_hbm,
              edst_v, pk_v, fids_v, fdst_v, rows_v, accm_v, s_sh, esem, gsem,
              ssem):
    cid = lax.axis_index("c")
    sid = lax.axis_index("s")
    wid = cid * NS + sid
    lo = wid * npt
    sloc = sid * npt
    cbase = cid * NS * npt
    n_blk = n_edges // EDGE_BLK
    n_ch = (NS * npt) // L
    lanes = lax.iota(jnp.int32, L)

    zf = jnp.zeros((L,), jnp.float32)
    zi = jnp.zeros((L,), jnp.int32)

    def zero_acc(i, _):
        accm_v[pl.ds(i * L, L)] = zf
        return 0
    lax.fori_loop(0, (npt + 1) * (H // L), zero_acc, 0)

    def zero_pk(i, _):
        pk_v[pl.ds(i * L, L)] = zi
        return 0
    lax.fori_loop(0, PK_CAP // L, zero_pk, 0)
    for s_ in range(NSLOT):
        for g in range(GB // L):
            fids_v[s_, pl.ds(g * L, L)] = zi
            fdst_v[s_, pl.ds(g * L, L)] = zi

    def zero_rows(i, _):
        for k in range(H // L):
            rows_v[i, pl.ds(k * L, L)] = zf
        return 0
    lax.fori_loop(0, L, zero_rows, 0)

    def zero_chunk(i, _):
        c = sid + NS * i

        @pl.when(c < n_ch)
        def _():
            pltpu.sync_copy(rows_v.at[pl.ds(0, L), :],
                            s_sh.at[pl.ds(c * L, L), :])
        return 0
    lax.fori_loop(0, (n_ch + NS - 1) // NS, zero_chunk, 0)
    plsc.subcore_barrier()

    def edge_dma(b):
        par = b % 2
        return pltpu.make_async_copy(
            edst_hbm.at[pl.ds(b * EDGE_BLK, EDGE_BLK)], edst_v.at[par],
            esem.at[par])

    def fill_slot(nf):
        s_ = nf % NSLOT
        for g in range(GB // L):
            pk = pk_v[pl.ds(g * L, L)]
            fids_v[s_, pl.ds(g * L, L)] = pk & (2**19 - 1)
            fdst_v[s_, pl.ds(g * L, L)] = (pk >> 19) + sloc
        for g in range((PK_CAP - GB) // L):
            pk_v[pl.ds(g * L, L)] = pk_v[pl.ds(GB + g * L, L)]

    def gather_start(nf):
        s_ = nf % NSLOT
        pltpu.async_copy(x_hbm.at[fids_v.at[s_]],
                         rows_v.at[pl.ds(s_ * GB, GB), :], gsem.at[s_])

    def finish(nf, nb):
        s_ = nf % NSLOT
        pltpu.make_async_copy(x_hbm.at[fids_v.at[s_]],
                              rows_v.at[pl.ds(s_ * GB, GB), :],
                              gsem.at[s_]).wait()

        def zrow(j, _):
            for k in range(H // L):
                rows_v[s_ * GB + j, pl.ds(k * L, L)] = zf
            return 0
        lax.fori_loop(nb, GB, zrow, 0)

        def acc_grp(jg, _):
            dg = fdst_v[s_, pl.ds(jg * L, L)]
            dvec = jnp.where(lanes + jg * L < nb, dg - sloc, npt)
            for k16 in range(L):
                off = dvec[k16] * H
                j = s_ * GB + jg * L + k16
                for k in range(H // L):
                    r = rows_v[j, pl.ds(k * L, L)]
                    cur = accm_v[pl.ds(off + k * L, L)]
                    accm_v[pl.ds(off + k * L, L)] = jnp.maximum(cur, r)
            return 0
        lax.fori_loop(0, GB // L, acc_grp, 0)
        pltpu.async_copy(rows_v.at[pl.ds(s_ * GB, GB), :],
                         s_sh.at[fdst_v.at[s_]], ssem.at[s_], add=True)

    def scat_wait(s_):
        pltpu.make_async_copy(rows_v.at[pl.ds(s_ * GB, GB), :],
                              s_sh.at[fdst_v.at[s_]], ssem.at[s_]).wait()

    edge_dma(0).start()

    def blk_body(b, carry):
        par_b = b % 2
        edge_dma(b).wait()

        @pl.when(b + 1 < n_blk)
        def _():
            edge_dma(b + 1).start()

        base = b * EDGE_BLK

        def chunk(c, carry):
            cnt, nf = carry
            dst = edst_v[par_b, pl.ds(c * L, L)]
            msk = (dst >= lo) & (dst < lo + npt)
            pk = ((dst - lo) << 19) | (lanes + (base + c * L))
            plsc.store_compressed(pk_v.at[pl.ds(cnt, L)], pk, mask=msk)
            cnt = cnt + plsc.all_reduce_population_count(msk)[0]

            def flush(args):
                cnt, nf = args

                @pl.when(nf >= NSLOT)
                def _():
                    scat_wait(nf % NSLOT)
                fill_slot(nf)
                gather_start(nf)

                @pl.when(nf >= 2)
                def _():
                    finish(nf - 2, GB)
                return (cnt - GB, nf + 1)
            return lax.cond(cnt >= GB, flush, lambda a: a, (cnt, nf))
        return lax.fori_loop(0, EDGE_BLK // L, chunk, carry)

    cnt, nf = lax.fori_loop(0, n_blk, blk_body,
                            (jnp.int32(0), jnp.int32(0)))

    @pl.when(nf >= 2)
    def _():
        finish(nf - 2, GB)

    @pl.when(nf >= 1)
    def _():
        finish(nf - 1, GB)

    @pl.when(cnt > 0)
    def _():
        @pl.when(nf >= NSLOT)
        def _():
            scat_wait(nf % NSLOT)
        fill_slot(nf)
        gather_start(nf)
        finish(nf, cnt)

    tf = jnp.where(cnt > 0, nf + 1, nf)
    for k in range(1, NSLOT + 1):
        @pl.when(tf >= k)
        def _(k=k):
            scat_wait((tf - k) % NSLOT)

    plsc.subcore_barrier()
    pltpu.sync_copy(accm_v.at[pl.ds(0, npt * H)],
                    m_hbm.at[pl.ds(lo * H, npt * H)])

    def out_chunk(i, _):
        c = sid + NS * i

        @pl.when(c < n_ch)
        def _():
            pltpu.sync_copy(s_sh.at[pl.ds(c * L, L), :],
                            s_hbm.at[pl.ds(cbase + c * L, L), :])
        return 0
    lax.fori_loop(0, (n_ch + NS - 1) // NS, out_chunk, 0)


def _segment_summax(x, edge_dst, n_pad):
    npt = n_pad // NW
    n_edges = x.shape[0]
    mesh = plsc.VectorSubcoreMesh(core_axis_name="c", subcore_axis_name="s",
                                  num_cores=NC, num_subcores=NS)
    body = functools.partial(_seg_body, npt, n_pad, n_edges)
    m_flat, s = pl.kernel(
        body,
        out_type=[jax.ShapeDtypeStruct((n_pad * H,), jnp.float32),
                  jax.ShapeDtypeStruct((n_pad, H), jnp.float32)],
        mesh=mesh,
        scratch_types=[
            pltpu.VMEM((2, EDGE_BLK), jnp.int32),
            pltpu.VMEM((PK_CAP,), jnp.int32),
            pltpu.VMEM((NSLOT, GB), jnp.int32),
            pltpu.VMEM((NSLOT, GB), jnp.int32),
            pltpu.VMEM((NSLOT * GB, H), jnp.float32),
            pltpu.VMEM(((npt + 1) * H,), jnp.float32),
            pltpu.VMEM_SHARED((NS * npt, H), jnp.float32),
            pltpu.SemaphoreType.DMA((2,)),
            pltpu.SemaphoreType.DMA((NSLOT,)),
            pltpu.SemaphoreType.DMA((NSLOT,)),
        ],
        compiler_params=pltpu.CompilerParams(needs_layout_passes=False),
    )(x, edge_dst)
    return m_flat.reshape(n_pad, H), s


def _gates_kernel(m0_ref, s0_ref, m1_ref, s1_ref, ia_ref, w1_ref, w2_ref,
                  w3_ref, wif_ref, bif_ref, wib_ref, bib_ref,
                  h0_ref, gf_ref, gb_ref):
    mg = (s0_ref[...] + s1_ref[...]) * jnp.maximum(m0_ref[...], m1_ref[...])
    ia = ia_ref[...]
    f = ia + 2.0 * mg
    am = ((_dot16(mg, w1_ref[...]) + _dot16(f, w2_ref[...]))
          + _dot16(ia, w3_ref[...]))
    t = am.shape[0]
    h0_ref[...] = jnp.max(am, axis=0).reshape(1, 1, H)
    gf = _dot16(am, wif_ref[...]) + bif_ref[...]
    gb = _dot16(am, wib_ref[...]) + bib_ref[...]
    gf_ref[...] = gf.reshape(t, 1, 1, 3 * H)
    gb_ref[...] = gb.reshape(t, 1, 1, 3 * H)


def _gru_kernel(n_per, gf_ref, gb_ref, h0_ref, whf_ref, bhf_ref,
                whb_ref, bhb_ref, wof_ref, wob_ref, bo_ref, o_ref):
    b = h0_ref.shape[0]
    h0 = h0_ref[...]
    zero = jnp.zeros((b, H), jnp.float32)

    def gru_step(g, h, wh_ref, bh_ref):
        gh = _dot16(h, wh_ref[...]) + bh_ref[...]
        r = jax.nn.sigmoid(g[:, :H] + gh[:, :H])
        z = jax.nn.sigmoid(g[:, H:2 * H] + gh[:, H:2 * H])
        n = jnp.tanh(g[:, 2 * H:] + r * gh[:, 2 * H:])
        return (1.0 - z) * n + z * h

    def step(t, carry):
        hf, hb, sf, sb = carry
        hf = gru_step(gf_ref[t], hf, whf_ref, bhf_ref)
        hb = gru_step(gb_ref[n_per - 1 - t], hb, whb_ref, bhb_ref)
        return (hf, hb, sf + hf, sb + hb)

    _, _, sf, sb = lax.fori_loop(0, n_per, step, (h0, h0, zero, zero))
    inv = 1.0 / n_per
    emb = (_dot16(sf * inv, wof_ref[...]) + _dot16(sb * inv, wob_ref[...])
           + bo_ref[...])
    o_ref[...] = jnp.maximum(emb, 0.0)


def kernel(f_atoms, f_bonds, edge_src, edge_dst, W_i_atom, W_i_bond, W_h_0, W_h_1,
           W_lr, W_o, b_o, gru_Wih_f, gru_Whh_f, gru_bih_f, gru_bhh_f,
           gru_Wih_b, gru_Whh_b, gru_bih_b, gru_bhh_b):
    n_nodes = f_atoms.shape[0]
    n_graphs = 50
    n_per = n_nodes // n_graphs
    n_pad = ((n_nodes + NW - 1) // NW) * NW

    ia = _mm_relu(f_atoms, W_i_atom.T, 1000)
    n_edges = f_bonds.shape[0]
    eh = n_edges // 2
    dst32 = edge_dst.astype(jnp.int32)
    x0 = _mm_relu(f_bonds[:eh], W_i_bond.T, 3200)
    x1 = _mm_relu(f_bonds[eh:], W_i_bond.T, 3200)

    m0, s0 = _segment_summax(x0, dst32[:eh], n_pad)
    m1, s1 = _segment_summax(x1, dst32[eh:], n_pad)

    h0, gf, gb = pl.pallas_call(
        _gates_kernel,
        grid=(n_graphs,),
        in_specs=[
            pl.BlockSpec((n_per, H), lambda i: (i, 0)),
            pl.BlockSpec((n_per, H), lambda i: (i, 0)),
            pl.BlockSpec((n_per, H), lambda i: (i, 0)),
            pl.BlockSpec((n_per, H), lambda i: (i, 0)),
            pl.BlockSpec((n_per, H), lambda i: (i, 0)),
            pl.BlockSpec((H, H), lambda i: (0, 0)),
            pl.BlockSpec((H, H), lambda i: (0, 0)),
            pl.BlockSpec((H, H), lambda i: (0, 0)),
            pl.BlockSpec((H, 3 * H), lambda i: (0, 0)),
            pl.BlockSpec((1, 3 * H), lambda i: (0, 0)),
            pl.BlockSpec((H, 3 * H), lambda i: (0, 0)),
            pl.BlockSpec((1, 3 * H), lambda i: (0, 0)),
        ],
        out_specs=[
            pl.BlockSpec((1, 1, H), lambda i: (i, 0, 0)),
            pl.BlockSpec((n_per, 1, 1, 3 * H), lambda i: (0, i, 0, 0)),
            pl.BlockSpec((n_per, 1, 1, 3 * H), lambda i: (0, i, 0, 0)),
        ],
        out_shape=[
            jax.ShapeDtypeStruct((n_graphs, 1, H), jnp.float32),
            jax.ShapeDtypeStruct((n_per, n_graphs, 1, 3 * H), jnp.float32),
            jax.ShapeDtypeStruct((n_per, n_graphs, 1, 3 * H), jnp.float32),
        ],
    )(m0, s0, m1, s1, ia, W_lr[:, :H].T, W_lr[:, H:2 * H].T, W_lr[:, 2 * H:].T,
      gru_Wih_f.T, gru_bih_f.reshape(1, 3 * H),
      gru_Wih_b.T, gru_bih_b.reshape(1, 3 * H))

    out = pl.pallas_call(
        functools.partial(_gru_kernel, n_per),
        out_shape=jax.ShapeDtypeStruct((n_graphs, H), jnp.float32),
    )(gf.reshape(n_per, n_graphs, 3 * H), gb.reshape(n_per, n_graphs, 3 * H),
      h0.reshape(n_graphs, H), gru_Whh_f.T, gru_bhh_f.reshape(1, 3 * H),
      gru_Whh_b.T, gru_bhh_b.reshape(1, 3 * H),
      W_o[:, :H].T, W_o[:, H:].T, b_o.reshape(1, H))
    return out


# speedup vs baseline: 1.4880x; 1.0459x over previous
import functools

import jax
import jax.numpy as jnp
from jax import lax
from jax.experimental import pallas as pl
from jax.experimental.pallas import tpu as pltpu
from jax.experimental.pallas import tpu_sc as plsc

H = 128
NC, NS, L = 2, 16, 16
NW = NC * NS
EDGE_BLK = 3200
GB = 96
NSLOT = 3
PK_CAP = 128


def _dot16(a, w):
    return jnp.dot(a.astype(jnp.bfloat16), w.astype(jnp.bfloat16),
                   preferred_element_type=jnp.float32)


def _mm_relu_kernel(a_ref, w_ref, o_ref):
    o_ref[...] = jnp.maximum(_dot16(a_ref[...], w_ref[...]), 0.0)


def _mm_relu(a, w_t, blk, r0=0, m_rows=None):
    m, k = a.shape
    n = w_t.shape[1]
    if m_rows is None:
        m_rows = m
    b0 = r0 // blk
    return pl.pallas_call(
        _mm_relu_kernel,
        grid=(m_rows // blk,),
        in_specs=[pl.BlockSpec((blk, k), lambda i: (i + b0, 0)),
                  pl.BlockSpec((k, n), lambda i: (0, 0))],
        out_specs=pl.BlockSpec((blk, n), lambda i: (i, 0)),
        out_shape=jax.ShapeDtypeStruct((m_rows, n), jnp.float32),
    )(a, w_t)


def _seg_body(npt, n_pad, n_edges, x_hbm, edst_hbm, m_hbm, s_hbm,
              edst_v, pk_v, fids_v, fdst_v, rows_v, accm_v, s_sh, esem, gsem,
              ssem):
    cid = lax.axis_index("c")
    sid = lax.axis_index("s")
    wid = cid * NS + sid
    lo = wid * npt
    sloc = sid * npt
    cbase = cid * NS * npt
    n_blk = n_edges // EDGE_BLK
    n_ch = (NS * npt) // L
    lanes = lax.iota(jnp.int32, L)

    zf = jnp.zeros((L,), jnp.float32)
    zi = jnp.zeros((L,), jnp.int32)

    def zero_acc(i, _):
        accm_v[pl.ds(i * L, L)] = zf
        return 0
    lax.fori_loop(0, (npt + 1) * (H // L), zero_acc, 0)

    def zero_pk(i, _):
        pk_v[pl.ds(i * L, L)] = zi
        return 0
    lax.fori_loop(0, PK_CAP // L, zero_pk, 0)
    for s_ in range(NSLOT):
        for g in range(GB // L):
            fids_v[s_, pl.ds(g * L, L)] = zi
            fdst_v[s_, pl.ds(g * L, L)] = zi

    def zero_rows(i, _):
        for k in range(H // L):
            rows_v[i, pl.ds(k * L, L)] = zf
        return 0
    lax.fori_loop(0, L, zero_rows, 0)

    def zero_chunk(i, _):
        c = sid + NS * i

        @pl.when(c < n_ch)
        def _():
            pltpu.sync_copy(rows_v.at[pl.ds(0, L), :],
                            s_sh.at[pl.ds(c * L, L), :])
        return 0
    lax.fori_loop(0, (n_ch + NS - 1) // NS, zero_chunk, 0)
    plsc.subcore_barrier()

    def edge_dma(b):
        par = b % 2
        return pltpu.make_async_copy(
            edst_hbm.at[pl.ds(b * EDGE_BLK, EDGE_BLK)], edst_v.at[par],
            esem.at[par])

    def fill_slot(nf):
        s_ = nf % NSLOT
        for g in range(GB // L):
            pk = pk_v[pl.ds(g * L, L)]
            fids_v[s_, pl.ds(g * L, L)] = pk & (2**19 - 1)
            fdst_v[s_, pl.ds(g * L, L)] = (pk >> 19) + sloc
        for g in range((PK_CAP - GB) // L):
            pk_v[pl.ds(g * L, L)] = pk_v[pl.ds(GB + g * L, L)]

    def gather_start(nf):
        s_ = nf % NSLOT
        pltpu.async_copy(x_hbm.at[fids_v.at[s_]],
                         rows_v.at[pl.ds(s_ * GB, GB), :], gsem.at[s_])

    def finish(nf, nb):
        s_ = nf % NSLOT
        pltpu.make_async_copy(x_hbm.at[fids_v.at[s_]],
                              rows_v.at[pl.ds(s_ * GB, GB), :],
                              gsem.at[s_]).wait()

        def zrow(j, _):
            for k in range(H // L):
                rows_v[s_ * GB + j, pl.ds(k * L, L)] = zf
            return 0
        lax.fori_loop(nb, GB, zrow, 0)

        def acc_grp(jg, _):
            dg = fdst_v[s_, pl.ds(jg * L, L)]
            dvec = jnp.where(lanes + jg * L < nb, dg - sloc, npt)
            for k16 in range(L):
                off = dvec[k16] * H
                j = s_ * GB + jg * L + k16
                for k in range(H // L):
                    r = rows_v[j, pl.ds(k * L, L)]
                    cur = accm_v[pl.ds(off + k * L, L)]
                    accm_v[pl.ds(off + k * L, L)] = jnp.maximum(cur, r)
            return 0
        lax.fori_loop(0, GB // L, acc_grp, 0)
        pltpu.async_copy(rows_v.at[pl.ds(s_ * GB, GB), :],
                         s_sh.at[fdst_v.at[s_]], ssem.at[s_], add=True)

    def scat_wait(s_):
        pltpu.make_async_copy(rows_v.at[pl.ds(s_ * GB, GB), :],
                              s_sh.at[fdst_v.at[s_]], ssem.at[s_]).wait()

    edge_dma(0).start()

    def blk_body(b, carry):
        par_b = b % 2
        edge_dma(b).wait()

        @pl.when(b + 1 < n_blk)
        def _():
            edge_dma(b + 1).start()

        base = b * EDGE_BLK

        def chunk(c, carry):
            cnt, nf = carry
            dst = edst_v[par_b, pl.ds(c * L, L)]
            msk = (dst >= lo) & (dst < lo + npt)
            pk = ((dst - lo) << 19) | (lanes + (base + c * L))
            plsc.store_compressed(pk_v.at[pl.ds(cnt, L)], pk, mask=msk)
            cnt = cnt + plsc.all_reduce_population_count(msk)[0]

            def flush(args):
                cnt, nf = args

                @pl.when(nf >= NSLOT)
                def _():
                    scat_wait(nf % NSLOT)
                fill_slot(nf)
                gather_start(nf)

                @pl.when(nf >= 2)
                def _():
                    finish(nf - 2, GB)
                return (cnt - GB, nf + 1)
            return lax.cond(cnt >= GB, flush, lambda a: a, (cnt, nf))
        return lax.fori_loop(0, EDGE_BLK // L, chunk, carry)

    cnt, nf = lax.fori_loop(0, n_blk, blk_body,
                            (jnp.int32(0), jnp.int32(0)))

    @pl.when(nf >= 2)
    def _():
        finish(nf - 2, GB)

    @pl.when(nf >= 1)
    def _():
        finish(nf - 1, GB)

    @pl.when(cnt > 0)
    def _():
        @pl.when(nf >= NSLOT)
        def _():
            scat_wait(nf % NSLOT)
        fill_slot(nf)
        gather_start(nf)
        finish(nf, cnt)

    tf = jnp.where(cnt > 0, nf + 1, nf)
    for k in range(1, NSLOT + 1):
        @pl.when(tf >= k)
        def _(k=k):
            scat_wait((tf - k) % NSLOT)

    plsc.subcore_barrier()
    pltpu.sync_copy(accm_v.at[pl.ds(0, npt * H)],
                    m_hbm.at[pl.ds(lo * H, npt * H)])

    def out_chunk(i, _):
        c = sid + NS * i

        @pl.when(c < n_ch)
        def _():
            pltpu.sync_copy(s_sh.at[pl.ds(c * L, L), :],
                            s_hbm.at[pl.ds(cbase + c * L, L), :])
        return 0
    lax.fori_loop(0, (n_ch + NS - 1) // NS, out_chunk, 0)


def _segment_summax(x, edge_dst, n_pad):
    npt = n_pad // NW
    n_edges = x.shape[0]
    mesh = plsc.VectorSubcoreMesh(core_axis_name="c", subcore_axis_name="s",
                                  num_cores=NC, num_subcores=NS)
    body = functools.partial(_seg_body, npt, n_pad, n_edges)
    m_flat, s = pl.kernel(
        body,
        out_type=[jax.ShapeDtypeStruct((n_pad * H,), jnp.float32),
                  jax.ShapeDtypeStruct((n_pad, H), jnp.float32)],
        mesh=mesh,
        scratch_types=[
            pltpu.VMEM((2, EDGE_BLK), jnp.int32),
            pltpu.VMEM((PK_CAP,), jnp.int32),
            pltpu.VMEM((NSLOT, GB), jnp.int32),
            pltpu.VMEM((NSLOT, GB), jnp.int32),
            pltpu.VMEM((NSLOT * GB, H), jnp.float32),
            pltpu.VMEM(((npt + 1) * H,), jnp.float32),
            pltpu.VMEM_SHARED((NS * npt, H), jnp.float32),
            pltpu.SemaphoreType.DMA((2,)),
            pltpu.SemaphoreType.DMA((NSLOT,)),
            pltpu.SemaphoreType.DMA((NSLOT,)),
        ],
        compiler_params=pltpu.CompilerParams(needs_layout_passes=False),
    )(x, edge_dst)
    return m_flat.reshape(n_pad, H), s


def _gates_kernel(m0_ref, s0_ref, m1_ref, s1_ref, ia_ref, w1_ref, w2_ref,
                  w3_ref, wif_ref, bif_ref, wib_ref, bib_ref,
                  h0_ref, gf_ref, gb_ref):
    mg = (s0_ref[...] + s1_ref[...]) * jnp.maximum(m0_ref[...], m1_ref[...])
    ia = ia_ref[...]
    f = ia + 2.0 * mg
    am = ((_dot16(mg, w1_ref[...]) + _dot16(f, w2_ref[...]))
          + _dot16(ia, w3_ref[...]))
    t = am.shape[0]
    h0_ref[...] = jnp.max(am, axis=0).reshape(1, 1, H)
    gf = _dot16(am, wif_ref[...]) + bif_ref[...]
    gb = _dot16(am, wib_ref[...]) + bib_ref[...]
    gf_ref[...] = gf.reshape(t, 1, 1, 3 * H)
    gb_ref[...] = gb.reshape(t, 1, 1, 3 * H)


def _gru_kernel(n_per, gf_ref, gb_ref, h0_ref, whf_ref, bhf_ref,
                whb_ref, bhb_ref, wof_ref, wob_ref, bo_ref, o_ref):
    b = h0_ref.shape[0]
    h0 = h0_ref[...]
    zero = jnp.zeros((b, H), jnp.float32)

    def gru_step(g, h, wh_ref, bh_ref):
        gh = _dot16(h, wh_ref[...]) + bh_ref[...]
        r = jax.nn.sigmoid(g[:, :H] + gh[:, :H])
        z = jax.nn.sigmoid(g[:, H:2 * H] + gh[:, H:2 * H])
        n = jnp.tanh(g[:, 2 * H:] + r * gh[:, 2 * H:])
        return (1.0 - z) * n + z * h

    def step(t, carry):
        hf, hb, sf, sb = carry
        hf = gru_step(gf_ref[t], hf, whf_ref, bhf_ref)
        hb = gru_step(gb_ref[n_per - 1 - t], hb, whb_ref, bhb_ref)
        return (hf, hb, sf + hf, sb + hb)

    _, _, sf, sb = lax.fori_loop(0, n_per, step, (h0, h0, zero, zero))
    inv = 1.0 / n_per
    emb = (_dot16(sf * inv, wof_ref[...]) + _dot16(sb * inv, wob_ref[...])
           + bo_ref[...])
    o_ref[...] = jnp.maximum(emb, 0.0)


def kernel(f_atoms, f_bonds, edge_src, edge_dst, W_i_atom, W_i_bond, W_h_0, W_h_1,
           W_lr, W_o, b_o, gru_Wih_f, gru_Whh_f, gru_bih_f, gru_bhh_f,
           gru_Wih_b, gru_Whh_b, gru_bih_b, gru_bhh_b):
    n_nodes = f_atoms.shape[0]
    n_graphs = 50
    n_per = n_nodes // n_graphs
    n_pad = ((n_nodes + NW - 1) // NW) * NW

    ia = _mm_relu(f_atoms, W_i_atom.T, 1000)
    n_edges = f_bonds.shape[0]
    eh = n_edges // 2
    dst32 = edge_dst.astype(jnp.int32)
    fb16 = f_bonds.astype(jnp.bfloat16)
    wb16 = W_i_bond.T.astype(jnp.bfloat16)
    x0 = _mm_relu(fb16, wb16, 3200, r0=0, m_rows=eh)
    x1 = _mm_relu(fb16, wb16, 3200, r0=eh, m_rows=eh)

    m0, s0 = _segment_summax(x0, dst32[:eh], n_pad)
    m1, s1 = _segment_summax(x1, dst32[eh:], n_pad)

    h0, gf, gb = pl.pallas_call(
        _gates_kernel,
        grid=(n_graphs,),
        in_specs=[
            pl.BlockSpec((n_per, H), lambda i: (i, 0)),
            pl.BlockSpec((n_per, H), lambda i: (i, 0)),
            pl.BlockSpec((n_per, H), lambda i: (i, 0)),
            pl.BlockSpec((n_per, H), lambda i: (i, 0)),
            pl.BlockSpec((n_per, H), lambda i: (i, 0)),
            pl.BlockSpec((H, H), lambda i: (0, 0)),
            pl.BlockSpec((H, H), lambda i: (0, 0)),
            pl.BlockSpec((H, H), lambda i: (0, 0)),
            pl.BlockSpec((H, 3 * H), lambda i: (0, 0)),
            pl.BlockSpec((1, 3 * H), lambda i: (0, 0)),
            pl.BlockSpec((H, 3 * H), lambda i: (0, 0)),
            pl.BlockSpec((1, 3 * H), lambda i: (0, 0)),
        ],
        out_specs=[
            pl.BlockSpec((1, 1, H), lambda i: (i, 0, 0)),
            pl.BlockSpec((n_per, 1, 1, 3 * H), lambda i: (0, i, 0, 0)),
            pl.BlockSpec((n_per, 1, 1, 3 * H), lambda i: (0, i, 0, 0)),
        ],
        out_shape=[
            jax.ShapeDtypeStruct((n_graphs, 1, H), jnp.float32),
            jax.ShapeDtypeStruct((n_per, n_graphs, 1, 3 * H), jnp.float32),
            jax.ShapeDtypeStruct((n_per, n_graphs, 1, 3 * H), jnp.float32),
        ],
    )(m0, s0, m1, s1, ia, W_lr[:, :H].T, W_lr[:, H:2 * H].T, W_lr[:, 2 * H:].T,
      gru_Wih_f.T, gru_bih_f.reshape(1, 3 * H),
      gru_Wih_b.T, gru_bih_b.reshape(1, 3 * H))

    out = pl.pallas_call(
        functools.partial(_gru_kernel, n_per),
        out_shape=jax.ShapeDtypeStruct((n_graphs, H), jnp.float32),
    )(gf.reshape(n_per, n_graphs, 3 * H), gb.reshape(n_per, n_graphs, 3 * H),
      h0.reshape(n_graphs, H), gru_Whh_f.T, gru_bhh_f.reshape(1, 3 * H),
      gru_Whh_b.T, gru_bhh_b.reshape(1, 3 * H),
      W_o[:, :H].T, W_o[:, H:].T, b_o.reshape(1, H))
    return out
